# baseline (speedup 1.0000x reference)
import functools

import jax
import jax.numpy as jnp
from jax import lax
from jax.experimental import pallas as pl
from jax.experimental.pallas import tpu as pltpu
from jax.experimental.pallas import tpu_sc as plsc

_N_ATOMS = 100000
_N_SEL = 50000
_D = 128
_DC = 3
_DCP = 16
_CHUNK = 80
_NCHUNK = _N_SEL // _CHUNK
_NW = 32
_MAXC = 20
_BIG = 17
_NB = 6
_LAG = 3

_mesh = plsc.VectorSubcoreMesh(core_axis_name="c", subcore_axis_name="s")


def _worker_span():
    w = lax.axis_index("s") * 2 + lax.axis_index("c")
    cnt = jnp.where(w < _BIG, _MAXC, _MAXC - 1)
    start_chunk = w * (_MAXC - 1) + jnp.minimum(w, _BIG)
    return start_chunk * _CHUNK, cnt


def _load_indices(idx_hbm, idx_v, rbase, cnt):
    low = (_MAXC - 1) * _CHUNK
    pltpu.sync_copy(idx_hbm.at[pl.ds(rbase, low)], idx_v.at[pl.ds(0, low)])

    @pl.when(cnt == _MAXC)
    def _():
        pltpu.sync_copy(idx_hbm.at[pl.ds(rbase + low, _CHUNK)],
                        idx_v.at[pl.ds(low, _CHUNK)])


@functools.partial(
    pl.kernel,
    mesh=_mesh,
    out_type=(jax.ShapeDtypeStruct((_N_SEL, _D), jnp.float32),
              jax.ShapeDtypeStruct((_N_SEL, _DCP), jnp.float32)),
    scratch_types=(
        [pltpu.VMEM((_MAXC * _CHUNK,), jnp.int32),
         pltpu.VMEM((_NB, _CHUNK, _D), jnp.float32),
         pltpu.VMEM((_NB, _CHUNK, _DCP), jnp.float32)]
        + [pltpu.SemaphoreType.DMA] * (4 * _NB)
    ),
    compiler_params=pltpu.CompilerParams(use_tc_tiling_on_sc=False),
)
def _gather_both(idx_hbm, atm_hbm, crd_hbm, atm_out, crd_out,
                 idx_v, abufs, cbufs, *sems):
    ga, gc = sems[:_NB], sems[_NB:2 * _NB]
    wa, wc = sems[2 * _NB:3 * _NB], sems[3 * _NB:]
    rbase, cnt = _worker_span()
    _load_indices(idx_hbm, idx_v, rbase, cnt)

    def wait_gather(b):
        pltpu.make_async_copy(atm_hbm.at[pl.ds(0, _CHUNK)], abufs.at[b],
                              ga[b]).wait()
        pltpu.make_async_copy(crd_hbm.at[pl.ds(0, _CHUNK)], cbufs.at[b],
                              gc[b]).wait()

    def wait_wb(b):
        pltpu.make_async_copy(abufs.at[b], atm_out.at[pl.ds(0, _CHUNK)],
                              wa[b]).wait()
        pltpu.make_async_copy(cbufs.at[b], crd_out.at[pl.ds(0, _CHUNK)],
                              wc[b]).wait()

    for j in range(_MAXC + _LAG):
        jc = j - _LAG
        if 0 <= jc < _MAXC:
            b = jc % _NB

            @pl.when(jc < cnt)
            def _(jc=jc, b=b):
                wait_gather(b)
                dst = pl.ds(rbase + jc * _CHUNK, _CHUNK)
                pltpu.async_copy(abufs.at[b], atm_out.at[dst], wa[b])
                pltpu.async_copy(cbufs.at[b], crd_out.at[dst], wc[b])

        if j < _MAXC:
            b = j % _NB

            @pl.when(j < cnt)
            def _(j=j, b=b):
                if j >= _NB:
                    wait_wb(b)
                sel = idx_v.at[pl.ds(j * _CHUNK, _CHUNK)]
                pltpu.async_copy(atm_hbm.at[sel], abufs.at[b], ga[b])
                pltpu.async_copy(crd_hbm.at[sel], cbufs.at[b], gc[b])

    for b in range(_NB):
        wait_wb(b)


def kernel(ent, atm, coord, mol_feat):
    e = jnp.reshape(ent, (_N_SEL,)).astype(jnp.int32)
    atm2 = jnp.reshape(atm, (_N_ATOMS, _D))
    coord2 = jnp.pad(jnp.reshape(coord, (_N_ATOMS, _DC)),
                     ((0, 0), (0, _DCP - _DC)))
    atm_sel, coord_sel = _gather_both(e, atm2, coord2)
    return (atm_sel[None], coord_sel[None, :, :_DC], mol_feat)


# speedup vs baseline: 1.2414x; 1.2414x over previous
import functools

import jax
import jax.numpy as jnp
from jax import lax
from jax.experimental import pallas as pl
from jax.experimental.pallas import tpu as pltpu
from jax.experimental.pallas import tpu_sc as plsc

_N_ATOMS = 100000
_N_SEL = 50000
_D = 128
_DC = 3
_CHUNK = 80
_NCHUNK = _N_SEL // _CHUNK
_NW = 32
_MAXC = 20
_BIG = 17
_NB = 6
_LAG = 3

_mesh = plsc.VectorSubcoreMesh(core_axis_name="c", subcore_axis_name="s")


def _worker_span():
    w = lax.axis_index("s") * 2 + lax.axis_index("c")
    cnt = jnp.where(w < _BIG, _MAXC, _MAXC - 1)
    start_chunk = w * (_MAXC - 1) + jnp.minimum(w, _BIG)
    return start_chunk * _CHUNK, cnt


def _load_indices(idx_hbm, idx_v, rbase, cnt):
    low = (_MAXC - 1) * _CHUNK
    pltpu.sync_copy(idx_hbm.at[pl.ds(rbase, low)], idx_v.at[pl.ds(0, low)])

    @pl.when(cnt == _MAXC)
    def _():
        pltpu.sync_copy(idx_hbm.at[pl.ds(rbase + low, _CHUNK)],
                        idx_v.at[pl.ds(low, _CHUNK)])


@functools.partial(
    pl.kernel,
    mesh=_mesh,
    out_type=(jax.ShapeDtypeStruct((1, _N_SEL, _D), jnp.float32),
              jax.ShapeDtypeStruct((_N_SEL, _D), jnp.float32)),
    scratch_types=(
        [pltpu.VMEM((_MAXC * _CHUNK,), jnp.int32),
         pltpu.VMEM((_NB, _CHUNK, _D), jnp.float32),
         pltpu.VMEM((_NB, _CHUNK, _D), jnp.float32)]
        + [pltpu.SemaphoreType.DMA] * (4 * _NB)
    ),
)
def _gather_both(idx_hbm, atm_hbm, crd_hbm, atm_out, crd_out,
                 idx_v, abufs, cbufs, *sems):
    ga, gc = sems[:_NB], sems[_NB:2 * _NB]
    wa, wc = sems[2 * _NB:3 * _NB], sems[3 * _NB:]
    rbase, cnt = _worker_span()
    _load_indices(idx_hbm, idx_v, rbase, cnt)

    atm_t = atm_hbm.at[0]
    a_out = atm_out.at[0]
    c_out = crd_out

    def wait_gather(b):
        pltpu.make_async_copy(atm_t.at[pl.ds(0, _CHUNK)], abufs.at[b],
                              ga[b]).wait()
        pltpu.make_async_copy(crd_hbm.at[pl.ds(0, _CHUNK)], cbufs.at[b],
                              gc[b]).wait()

    def wait_wb(b):
        pltpu.make_async_copy(abufs.at[b], a_out.at[pl.ds(0, _CHUNK)],
                              wa[b]).wait()
        pltpu.make_async_copy(cbufs.at[b], c_out.at[pl.ds(0, _CHUNK)],
                              wc[b]).wait()

    for j in range(_MAXC + _LAG):
        jc = j - _LAG
        if 0 <= jc < _MAXC:
            b = jc % _NB

            @pl.when(jc < cnt)
            def _(jc=jc, b=b):
                wait_gather(b)
                dst = pl.ds(rbase + jc * _CHUNK, _CHUNK)
                pltpu.async_copy(abufs.at[b], a_out.at[dst], wa[b])
                pltpu.async_copy(cbufs.at[b], c_out.at[dst], wc[b])

        if j < _MAXC:
            b = j % _NB

            @pl.when(j < cnt)
            def _(j=j, b=b):
                if j >= _NB:
                    wait_wb(b)
                sel = idx_v.at[pl.ds(j * _CHUNK, _CHUNK)]
                pltpu.async_copy(atm_t.at[sel], abufs.at[b], ga[b])
                pltpu.async_copy(crd_hbm.at[sel], cbufs.at[b], gc[b])

    for b in range(_NB):
        wait_wb(b)


def kernel(ent, atm, coord, mol_feat):
    e = jnp.reshape(ent, (_N_SEL,)).astype(jnp.int32)
    crd128 = jnp.pad(jnp.reshape(coord, (_N_ATOMS, _DC)),
                     ((0, 0), (0, _D - _DC)))
    atm_sel, coord_sel = _gather_both(e, atm, crd128)
    return (atm_sel, coord_sel[None, :, :_DC], mol_feat)


# speedup vs baseline: 1.2421x; 1.0005x over previous
import functools

import jax
import jax.numpy as jnp
from jax import lax
from jax.experimental import pallas as pl
from jax.experimental.pallas import tpu as pltpu
from jax.experimental.pallas import tpu_sc as plsc

_N_ATOMS = 100000
_N_SEL = 50000
_D = 128
_DC = 3
_CHUNK = 80
_NCHUNK = _N_SEL // _CHUNK
_NW = 32
_MAXC = 20
_BIG = 17
_NB = 6
_LAG = 3

_mesh = plsc.VectorSubcoreMesh(core_axis_name="c", subcore_axis_name="s")


def _worker_span():
    w = lax.axis_index("s") * 2 + lax.axis_index("c")
    cnt = jnp.where(w < _BIG, _MAXC, _MAXC - 1)
    start_chunk = w * (_MAXC - 1) + jnp.minimum(w, _BIG)
    return start_chunk * _CHUNK, cnt


def _load_indices(idx_hbm, idx_v, rbase, cnt):
    low = (_MAXC - 1) * _CHUNK
    pltpu.sync_copy(idx_hbm.at[pl.ds(rbase, low)], idx_v.at[pl.ds(0, low)])

    @pl.when(cnt == _MAXC)
    def _():
        pltpu.sync_copy(idx_hbm.at[pl.ds(rbase + low, _CHUNK)],
                        idx_v.at[pl.ds(low, _CHUNK)])


@functools.partial(
    pl.kernel,
    mesh=_mesh,
    out_type=(jax.ShapeDtypeStruct((1, _N_SEL, _D), jnp.float32),
              jax.ShapeDtypeStruct((_N_SEL, _D), jnp.float32)),
    scratch_types=(
        [pltpu.VMEM((_MAXC * _CHUNK,), jnp.int32),
         pltpu.VMEM((_NB, _CHUNK, _D), jnp.float32),
         pltpu.VMEM((_NB, _CHUNK, _D), jnp.float32)]
        + [pltpu.SemaphoreType.DMA] * (4 * _NB)
    ),
)
def _gather_both(idx_hbm, atm_hbm, crd_hbm, atm_out, crd_out,
                 idx_v, abufs, cbufs, *sems):
    ga, gc = sems[:_NB], sems[_NB:2 * _NB]
    wa, wc = sems[2 * _NB:3 * _NB], sems[3 * _NB:]
    rbase, cnt = _worker_span()
    _load_indices(idx_hbm, idx_v, rbase, cnt)

    atm_t = atm_hbm.at[0]
    a_out = atm_out.at[0]
    c_out = crd_out

    def wait_gather(b):
        pltpu.make_async_copy(atm_t.at[pl.ds(0, _CHUNK)], abufs.at[b],
                              ga[b]).wait()
        pltpu.make_async_copy(crd_hbm.at[pl.ds(0, _CHUNK)], cbufs.at[b],
                              gc[b]).wait()

    def wait_wb(b):
        pltpu.make_async_copy(abufs.at[b], a_out.at[pl.ds(0, _CHUNK)],
                              wa[b]).wait()
        pltpu.make_async_copy(cbufs.at[b], c_out.at[pl.ds(0, _CHUNK)],
                              wc[b]).wait()

    for j in range(_MAXC + _LAG):
        jc = j - _LAG
        if 0 <= jc < _MAXC:
            b = jc % _NB

            @pl.when(jc < cnt)
            def _(jc=jc, b=b):
                wait_gather(b)
                dst = pl.ds(rbase + jc * _CHUNK, _CHUNK)
                pltpu.async_copy(abufs.at[b], a_out.at[dst], wa[b])
                pltpu.async_copy(cbufs.at[b], c_out.at[dst], wc[b])

        if j < _MAXC:
            b = j % _NB

            @pl.when(j < cnt)
            def _(j=j, b=b):
                if j >= _NB:
                    wait_wb(b)
                sel = idx_v.at[pl.ds(j * _CHUNK, _CHUNK)]
                pltpu.async_copy(atm_t.at[sel], abufs.at[b], ga[b])
                pltpu.async_copy(crd_hbm.at[sel], cbufs.at[b], gc[b])

    for b in range(_NB):
        wait_wb(b)


def kernel(ent, atm, coord, mol_feat):
    e = jnp.reshape(ent, (_N_SEL,)).astype(jnp.int32)
    crd128 = jnp.pad(jnp.reshape(coord, (_N_ATOMS, _DC)),
                     ((0, 0), (0, _D - _DC)))
    atm_sel, coord_sel = _gather_both(e, atm, crd128)
    coord_out = jnp.reshape(coord_sel[:, :_DC], (1, _N_SEL, _DC))
    return (atm_sel, coord_out, mol_feat)
